# Initial kernel scaffold; baseline (speedup 1.0000x reference)
import jax
import jax.numpy as jnp
from jax.experimental import pallas as pl


def kernel(x, t_embed, h_embed, w_embed):
    raise NotImplementedError("write your pallas kernel here")


# speedup vs baseline: 1.1438x; 1.1438x over previous
import jax
import jax.numpy as jnp
from jax.experimental import pallas as pl

T_DIM, H_DIM, W_DIM, EMBED_DIM = 16, 24, 24, 384
BATCH = 8

T_BLK = 4


def _body(t_ref, h_ref, w_ref, x_ref, o_ref):
    t = t_ref[0]
    h = h_ref[...]
    w = w_ref[...]
    th = t[:, None, :] * h[None, :, :]
    pos = th[:, :, None, :] * w[None, None, :, :]
    o_ref[...] = x_ref[...] + pos


def kernel(x, t_embed, h_embed, w_embed):
    bt = BATCH * T_DIM
    xr = x.reshape(bt, H_DIM, W_DIM, EMBED_DIM)
    tr = t_embed.reshape(T_DIM // T_BLK, T_BLK, EMBED_DIM)
    grid = (bt // T_BLK,)
    out = pl.pallas_call(
        _body,
        grid=grid,
        in_specs=[
            pl.BlockSpec((1, T_BLK, EMBED_DIM), lambda i: (i % (T_DIM // T_BLK), 0, 0)),
            pl.BlockSpec((H_DIM, EMBED_DIM), lambda i: (0, 0)),
            pl.BlockSpec((W_DIM, EMBED_DIM), lambda i: (0, 0)),
            pl.BlockSpec((T_BLK, H_DIM, W_DIM, EMBED_DIM), lambda i: (i, 0, 0, 0)),
        ],
        out_specs=pl.BlockSpec((T_BLK, H_DIM, W_DIM, EMBED_DIM), lambda i: (i, 0, 0, 0)),
        out_shape=jax.ShapeDtypeStruct((bt, H_DIM, W_DIM, EMBED_DIM), x.dtype),
    )(tr, h_embed, w_embed, xr)
    return out.reshape(x.shape)


# speedup vs baseline: 1.2039x; 1.0525x over previous
import jax
import jax.numpy as jnp
from jax.experimental import pallas as pl

T_DIM, H_DIM, W_DIM, EMBED_DIM = 16, 24, 24, 384
BATCH = 8

T_BLK = 8


def _body(t_ref, h_ref, w_ref, x_ref, o_ref):
    t = t_ref[0]
    h = h_ref[...]
    w = w_ref[...]
    th = t[:, None, :] * h[None, :, :]
    pos = th[:, :, None, :] * w[None, None, :, :]
    o_ref[...] = x_ref[...] + pos


def kernel(x, t_embed, h_embed, w_embed):
    bt = BATCH * T_DIM
    xr = x.reshape(bt, H_DIM, W_DIM, EMBED_DIM)
    tr = t_embed.reshape(T_DIM // T_BLK, T_BLK, EMBED_DIM)
    grid = (bt // T_BLK,)
    out = pl.pallas_call(
        _body,
        grid=grid,
        in_specs=[
            pl.BlockSpec((1, T_BLK, EMBED_DIM), lambda i: (i % (T_DIM // T_BLK), 0, 0)),
            pl.BlockSpec((H_DIM, EMBED_DIM), lambda i: (0, 0)),
            pl.BlockSpec((W_DIM, EMBED_DIM), lambda i: (0, 0)),
            pl.BlockSpec((T_BLK, H_DIM, W_DIM, EMBED_DIM), lambda i: (i, 0, 0, 0)),
        ],
        out_specs=pl.BlockSpec((T_BLK, H_DIM, W_DIM, EMBED_DIM), lambda i: (i, 0, 0, 0)),
        out_shape=jax.ShapeDtypeStruct((bt, H_DIM, W_DIM, EMBED_DIM), x.dtype),
    )(tr, h_embed, w_embed, xr)
    return out.reshape(x.shape)


# speedup vs baseline: 1.2192x; 1.0127x over previous
import jax
import jax.numpy as jnp
from jax.experimental import pallas as pl

T_DIM, H_DIM, W_DIM, EMBED_DIM = 16, 24, 24, 384
BATCH = 8

T_BLK = 16


def _body(t_ref, h_ref, w_ref, x_ref, o_ref):
    t = t_ref[0]
    h = h_ref[...]
    w = w_ref[...]
    th = t[:, None, :] * h[None, :, :]
    pos = th[:, :, None, :] * w[None, None, :, :]
    o_ref[...] = x_ref[...] + pos


def kernel(x, t_embed, h_embed, w_embed):
    bt = BATCH * T_DIM
    xr = x.reshape(bt, H_DIM, W_DIM, EMBED_DIM)
    tr = t_embed.reshape(T_DIM // T_BLK, T_BLK, EMBED_DIM)
    grid = (bt // T_BLK,)
    out = pl.pallas_call(
        _body,
        grid=grid,
        in_specs=[
            pl.BlockSpec((1, T_BLK, EMBED_DIM), lambda i: (i % (T_DIM // T_BLK), 0, 0)),
            pl.BlockSpec((H_DIM, EMBED_DIM), lambda i: (0, 0)),
            pl.BlockSpec((W_DIM, EMBED_DIM), lambda i: (0, 0)),
            pl.BlockSpec((T_BLK, H_DIM, W_DIM, EMBED_DIM), lambda i: (i, 0, 0, 0)),
        ],
        out_specs=pl.BlockSpec((T_BLK, H_DIM, W_DIM, EMBED_DIM), lambda i: (i, 0, 0, 0)),
        out_shape=jax.ShapeDtypeStruct((bt, H_DIM, W_DIM, EMBED_DIM), x.dtype),
    )(tr, h_embed, w_embed, xr)
    return out.reshape(x.shape)
